# baseline (speedup 1.0000x reference)
import functools

import jax
import jax.numpy as jnp
from jax import lax
from jax.experimental import pallas as pl
from jax.experimental.pallas import tpu as pltpu
from jax.experimental.pallas import tpu_sc as plsc

_S = 4096
_D = 4096
_NC = 2
_NS = 16
_NW = _NC * _NS
_ROWS_PER_W = _S // _NW
_CH = 8
_NCHUNK = _ROWS_PER_W // _CH
_LANES = 16

_mesh = plsc.VectorSubcoreMesh(core_axis_name="c", subcore_axis_name="s")


@functools.partial(
    pl.kernel,
    mesh=_mesh,
    out_type=jax.ShapeDtypeStruct((_S, _D), jnp.float32),
    scratch_types=[
        pltpu.VMEM((_CH, _D), jnp.float32),
        pltpu.VMEM((_CH, _D), jnp.float32),
    ],
)
def _sc_add(x_hbm, p_hbm, o_hbm, xv, pv):
    wid = lax.axis_index("s") * _NC + lax.axis_index("c")
    base = wid * _ROWS_PER_W

    def chunk_body(ci, carry):
        rb = base + ci * _CH
        pltpu.sync_copy(x_hbm.at[pl.ds(rb, _CH)], xv)
        pltpu.sync_copy(p_hbm.at[pl.ds(rb, _CH)], pv)

        def row_body(r, carry2):
            def vec_body(j, carry3):
                c = j * _LANES
                xv[r, pl.ds(c, _LANES)] = (
                    xv[r, pl.ds(c, _LANES)] + pv[r, pl.ds(c, _LANES)]
                )
                return carry3

            return lax.fori_loop(0, _D // _LANES, vec_body, carry2)

        lax.fori_loop(0, _CH, row_body, 0)
        pltpu.sync_copy(xv, o_hbm.at[pl.ds(rb, _CH)])
        return carry

    lax.fori_loop(0, _NCHUNK, chunk_body, 0)


def kernel(x, pos_table):
    return _sc_add(x, pos_table)


# speedup vs baseline: 2.7816x; 2.7816x over previous
import functools

import jax
import jax.numpy as jnp
from jax import lax
from jax.experimental import pallas as pl
from jax.experimental.pallas import tpu as pltpu
from jax.experimental.pallas import tpu_sc as plsc

_S = 4096
_D = 4096
_NC = 2
_NS = 16
_NW = _NC * _NS
_ROWS_PER_W = _S // _NW
_CH = 4
_NCHUNK = _ROWS_PER_W // _CH
_LANES = 16
_UNROLL = 8

_mesh = plsc.VectorSubcoreMesh(core_axis_name="c", subcore_axis_name="s")

_VBUF = pltpu.VMEM((_CH, _D), jnp.float32)


@functools.partial(
    pl.kernel,
    mesh=_mesh,
    out_type=jax.ShapeDtypeStruct((_S, _D), jnp.float32),
    scratch_types=[
        _VBUF, _VBUF, _VBUF,
        _VBUF, _VBUF, _VBUF,
        pltpu.SemaphoreType.DMA,
        pltpu.SemaphoreType.DMA,
        pltpu.SemaphoreType.DMA,
        pltpu.SemaphoreType.DMA,
    ],
)
def _sc_add(x_hbm, p_hbm, o_hbm, xv0, pv0, ov0, xv1, pv1, ov1,
            in0, in1, out0, out1):
    wid = lax.axis_index("s") * _NC + lax.axis_index("c")
    base = wid * _ROWS_PER_W
    xv = (xv0, xv1)
    pv = (pv0, pv1)
    ov = (ov0, ov1)
    ins = (in0, in1)
    outs = (out0, out1)

    def start_in(chunk, b):
        rb = base + chunk * _CH
        pltpu.async_copy(x_hbm.at[pl.ds(rb, _CH)], xv[b], ins[b])
        pltpu.async_copy(p_hbm.at[pl.ds(rb, _CH)], pv[b], ins[b])

    def wait_in(b):
        pltpu.make_async_copy(x_hbm.at[pl.ds(base, _CH)], xv[b], ins[b]).wait()
        pltpu.make_async_copy(p_hbm.at[pl.ds(base, _CH)], pv[b], ins[b]).wait()

    def start_out(chunk, b):
        rb = base + chunk * _CH
        pltpu.async_copy(ov[b], o_hbm.at[pl.ds(rb, _CH)], outs[b])

    def wait_out(b):
        pltpu.make_async_copy(
            ov[b], o_hbm.at[pl.ds(base, _CH)], outs[b]).wait()

    start_in(0, 0)
    start_in(1, 1)

    def group_body(g, carry):
        for b in range(2):
            chunk = 2 * g + b
            wait_in(b)

            @pl.when(chunk >= 2)
            def _():
                wait_out(b)

            for r in range(_CH):
                def vec_body(j, carry2):
                    c = j * (_LANES * _UNROLL)
                    for u in range(_UNROLL):
                        s = pl.ds(c + u * _LANES, _LANES)
                        ov[b][r, s] = xv[b][r, s] + pv[b][r, s]
                    return carry2

                lax.fori_loop(0, _D // (_LANES * _UNROLL), vec_body, 0)

            start_out(chunk, b)

            @pl.when(chunk + 2 < _NCHUNK)
            def _():
                start_in(chunk + 2, b)
        return carry

    lax.fori_loop(0, _NCHUNK // 2, group_body, 0)
    wait_out(0)
    wait_out(1)


def kernel(x, pos_table):
    return _sc_add(x, pos_table)
